# baseline (speedup 1.0000x reference)
import jax
import jax.numpy as jnp
from jax import lax
from jax.experimental import pallas as pl


_ROWS = 16384
_COLS = 50
_CLASSES = 256
_R2 = 6400
_C2 = 128
_BLOCK_R = 32


def _onehot_body(x_ref, out_ref):
    x = x_ref[...].astype(jnp.int32)
    classes = lax.broadcasted_iota(jnp.int32, (_BLOCK_R, _C2, _CLASSES), 2)
    out_ref[...] = (x[:, :, None] == classes).astype(jnp.float32)


def kernel(x):
    x2 = x.reshape(_R2, _C2).astype(jnp.int32)
    out = pl.pallas_call(
        _onehot_body,
        grid=(_R2 // _BLOCK_R,),
        in_specs=[pl.BlockSpec((_BLOCK_R, _C2), lambda i: (i, 0))],
        out_specs=pl.BlockSpec((_BLOCK_R, _C2, _CLASSES), lambda i: (i, 0, 0)),
        out_shape=jax.ShapeDtypeStruct((_R2, _C2, _CLASSES), jnp.float32),
    )(x2)
    return out.reshape(_ROWS, _COLS, _CLASSES)
